# baseline (speedup 1.0000x reference)
import functools

import jax
import jax.numpy as jnp
from jax import lax
from jax.experimental import pallas as pl
from jax.experimental.pallas import tpu as pltpu
from jax.experimental.pallas import tpu_sc as plsc

BATCH = 16384
EMBED_K = 32
NUM_ROWS = 1000000
NUM_WORKERS = 32
ROWS_PER_WORKER = BATCH // NUM_WORKERS
CHUNK = 128
NCHUNK = ROWS_PER_WORKER // CHUNK
GROUPS = ROWS_PER_WORKER // 16


def _sc_mf_body(uidx_hbm, vidx_hbm, wt_hbm, ht_hbm, out_hbm,
                uidx_v, vidx_v, u_vals, v_vals, out_v, sem):
    cid = lax.axis_index("c")
    sid = lax.axis_index("s")
    wid = sid * 2 + cid
    base = wid * ROWS_PER_WORKER

    row0 = wid * NCHUNK
    pltpu.sync_copy(uidx_hbm.at[pl.ds(row0, NCHUNK)], uidx_v)
    pltpu.sync_copy(vidx_hbm.at[pl.ds(row0, NCHUNK)], vidx_v)

    for k in range(EMBED_K):
        copies = []
        for j in range(NCHUNK):
            copies.append(pltpu.async_copy(
                wt_hbm.at[k].at[uidx_v.at[j]],
                u_vals.at[pl.ds(k * ROWS_PER_WORKER + j * CHUNK, CHUNK)],
                sem))
            copies.append(pltpu.async_copy(
                ht_hbm.at[k].at[vidx_v.at[j]],
                v_vals.at[pl.ds(k * ROWS_PER_WORKER + j * CHUNK, CHUNK)],
                sem))
        for c in copies:
            c.wait()

    def group_body(g, carry):
        acc = jnp.zeros((16,), jnp.float32)
        for k in range(EMBED_K):
            u = u_vals[pl.ds(k * ROWS_PER_WORKER + g * 16, 16)]
            v = v_vals[pl.ds(k * ROWS_PER_WORKER + g * 16, 16)]
            acc = acc + u * v
        out_v[pl.ds(g * 16, 16)] = acc
        return carry

    lax.fori_loop(0, GROUPS, group_body, 0)

    pltpu.sync_copy(out_v, out_hbm.at[pl.ds(base, ROWS_PER_WORKER)])


@functools.partial(
    pl.kernel,
    out_type=jax.ShapeDtypeStruct((BATCH,), jnp.float32),
    mesh=plsc.VectorSubcoreMesh(core_axis_name="c", subcore_axis_name="s"),
    compiler_params=pltpu.CompilerParams(
        needs_layout_passes=False, use_tc_tiling_on_sc=False),
    scratch_types=[
        pltpu.VMEM((NCHUNK, CHUNK), jnp.int32),
        pltpu.VMEM((NCHUNK, CHUNK), jnp.int32),
        pltpu.VMEM((EMBED_K * ROWS_PER_WORKER,), jnp.float32),
        pltpu.VMEM((EMBED_K * ROWS_PER_WORKER,), jnp.float32),
        pltpu.VMEM((ROWS_PER_WORKER,), jnp.float32),
        pltpu.SemaphoreType.DMA,
    ],
)
def _mf_sc(uidx_hbm, vidx_hbm, wt_hbm, ht_hbm, out_hbm,
           uidx_v, vidx_v, u_vals, v_vals, out_v, sem):
    _sc_mf_body(uidx_hbm, vidx_hbm, wt_hbm, ht_hbm, out_hbm,
                uidx_v, vidx_v, u_vals, v_vals, out_v, sem)


def kernel(x, W, H):
    uidx = x[:, 0].astype(jnp.int32).reshape(BATCH // CHUNK, CHUNK)
    vidx = x[:, 1].astype(jnp.int32).reshape(BATCH // CHUNK, CHUNK)
    return _mf_sc(uidx, vidx, W.T, H.T)


# speedup vs baseline: 21.1921x; 21.1921x over previous
import functools

import jax
import jax.numpy as jnp
from jax import lax
from jax.experimental import pallas as pl
from jax.experimental.pallas import tpu as pltpu
from jax.experimental.pallas import tpu_sc as plsc

BATCH = 16384
EMBED_K = 32
NUM_ROWS = 1000000
NUM_WORKERS = 32
ROWS_PER_WORKER = BATCH // NUM_WORKERS
GROUPS = ROWS_PER_WORKER // 16
LANE = 128


def _fetch_blocks(table_hbm, idxv, blocks, sem):
    copies = []
    for i in range(16):
        tcol = pl.multiple_of((idxv[i] // LANE) * LANE, LANE)
        copies.append(pltpu.async_copy(
            table_hbm.at[:, pl.ds(tcol, LANE)], blocks.at[i], sem))
    return copies


def _sc_mf_body(uidx_hbm, vidx_hbm, wt_hbm, ht_hbm, out_hbm,
                uidx_v, vidx_v, blocks, u_slab, out_v, sem):
    cid = lax.axis_index("c")
    sid = lax.axis_index("s")
    wid = sid * 2 + cid
    base = wid * ROWS_PER_WORKER

    pltpu.sync_copy(uidx_hbm.at[pl.ds(base, ROWS_PER_WORKER)], uidx_v)
    pltpu.sync_copy(vidx_hbm.at[pl.ds(base, ROWS_PER_WORKER)], vidx_v)

    iota = lax.iota(jnp.int32, 16)

    def group_body(g, carry):
        uvec = uidx_v[pl.ds(g * 16, 16)]
        ucols = jnp.bitwise_and(uvec, LANE - 1)
        for c in _fetch_blocks(wt_hbm, uvec, blocks, sem):
            c.wait()
        for k in range(EMBED_K):
            u = plsc.load_gather(
                blocks, [iota, jnp.full((16,), k, jnp.int32), ucols])
            u_slab[k, :] = u
        vvec = vidx_v[pl.ds(g * 16, 16)]
        vcols = jnp.bitwise_and(vvec, LANE - 1)
        for c in _fetch_blocks(ht_hbm, vvec, blocks, sem):
            c.wait()
        acc = jnp.zeros((16,), jnp.float32)
        for k in range(EMBED_K):
            v = plsc.load_gather(
                blocks, [iota, jnp.full((16,), k, jnp.int32), vcols])
            acc = acc + u_slab[k, :] * v
        out_v[pl.ds(g * 16, 16)] = acc
        return carry

    lax.fori_loop(0, GROUPS, group_body, 0)

    pltpu.sync_copy(out_v, out_hbm.at[pl.ds(base, ROWS_PER_WORKER)])


@functools.partial(
    pl.kernel,
    out_type=jax.ShapeDtypeStruct((BATCH,), jnp.float32),
    mesh=plsc.VectorSubcoreMesh(core_axis_name="c", subcore_axis_name="s"),
    compiler_params=pltpu.CompilerParams(
        needs_layout_passes=False, use_tc_tiling_on_sc=True),
    scratch_types=[
        pltpu.VMEM((ROWS_PER_WORKER,), jnp.int32),
        pltpu.VMEM((ROWS_PER_WORKER,), jnp.int32),
        pltpu.VMEM((16, EMBED_K, LANE), jnp.float32),
        pltpu.VMEM((EMBED_K, 16), jnp.float32),
        pltpu.VMEM((ROWS_PER_WORKER,), jnp.float32),
        pltpu.SemaphoreType.DMA,
    ],
)
def _mf_sc(uidx_hbm, vidx_hbm, wt_hbm, ht_hbm, out_hbm,
           uidx_v, vidx_v, blocks, u_slab, out_v, sem):
    _sc_mf_body(uidx_hbm, vidx_hbm, wt_hbm, ht_hbm, out_hbm,
                uidx_v, vidx_v, blocks, u_slab, out_v, sem)


def kernel(x, W, H):
    uidx = x[:, 0].astype(jnp.int32)
    vidx = x[:, 1].astype(jnp.int32)
    return _mf_sc(uidx, vidx, W.T, H.T)
